# Initial kernel scaffold; baseline (speedup 1.0000x reference)
import jax
import jax.numpy as jnp
from jax.experimental import pallas as pl


def kernel(embeddings, labels, mass, size_map):
    raise NotImplementedError("write your pallas kernel here")


# speedup vs baseline: 2.0290x; 2.0290x over previous
import jax
import jax.numpy as jnp
from jax import lax
from jax.experimental import pallas as pl
from jax.experimental.pallas import tpu as pltpu
from jax.experimental.pallas import tpu_sc as plsc

_ALPHA = 0.1
_N = 32768
_D = 128
_K = 256
_NC = 2
_NS = 16
_NW = _NC * _NS
_TPW = _N // _NW
_CHUNK = 128
_NCHUNK = _TPW // _CHUNK
_GPC = _CHUNK // 16
_COLS = 384


def _vec_sqrt(x):
    i = lax.bitcast_convert_type(x, jnp.int32)
    i = jnp.int32(0x5F3759DF) - lax.shift_right_logical(i, 1)
    y = lax.bitcast_convert_type(i, jnp.float32)
    half = x * 0.5
    for _ in range(3):
        y = y * (1.5 - half * y * y)
    return jnp.where(x == 0.0, 0.0, x * y)


def _sc_body(emb, labels, mass, zinit, out, lab_v, mass_v, ebuf, acc):
    cid = lax.axis_index("c")
    sid = lax.axis_index("s")
    wid = sid * _NC + cid
    tok0 = wid * _TPW

    pltpu.sync_copy(zinit, acc)
    pltpu.sync_copy(labels.at[pl.ds(tok0, _TPW)], lab_v)
    pltpu.sync_copy(mass.at[pl.ds(tok0, _TPW)], mass_v)

    iota16 = lax.iota(jnp.int32, 16)
    zeros16 = jnp.zeros((16,), jnp.float32)
    ones16 = jnp.ones((16,), jnp.float32)

    def _chunk(ci, carry):
        pltpu.sync_copy(
            emb.at[pl.ds((tok0 + ci * _CHUNK) * _D, _CHUNK * _D)], ebuf)

        def _group(g, c2):
            base = ci * _CHUNK + g * 16
            labv = lab_v[pl.ds(base, 16)]
            m = _vec_sqrt(mass_v[pl.ds(base, 16)])
            rowsb = (iota16 + g * 16) * _D
            sq = zeros16
            for d in range(_D):
                dvec = jnp.full((16,), d, jnp.int32)
                ed = plsc.load_gather(ebuf, [rowsb + d])
                plsc.addupdate_scatter(acc, [labv, dvec], ed * m)
                plsc.addupdate_scatter(acc, [labv, dvec + _D], ed)
                sq = sq + ed * ed
            plsc.addupdate_scatter(acc, [labv, jnp.full((16,), 256, jnp.int32)], m)
            plsc.addupdate_scatter(acc, [labv, jnp.full((16,), 257, jnp.int32)], sq)
            plsc.addupdate_scatter(acc, [labv, jnp.full((16,), 258, jnp.int32)], ones16)
            return c2

        lax.fori_loop(0, _GPC, _group, 0)
        return carry

    lax.fori_loop(0, _NCHUNK, _chunk, 0)

    pltpu.sync_copy(acc, out.at[wid])


_sc_stage1 = pl.kernel(
    _sc_body,
    out_type=jax.ShapeDtypeStruct((_NW, _K, _COLS), jnp.float32),
    mesh=plsc.VectorSubcoreMesh(
        core_axis_name="c", subcore_axis_name="s",
        num_cores=_NC, num_subcores=_NS),
    compiler_params=pltpu.CompilerParams(needs_layout_passes=False),
    scratch_types=[
        pltpu.VMEM((_TPW,), jnp.int32),
        pltpu.VMEM((_TPW,), jnp.float32),
        pltpu.VMEM((_CHUNK * _D,), jnp.float32),
        pltpu.VMEM((_K, _COLS), jnp.float32),
    ],
)


def _tc_body(stats_ref, size_ref, o_ref):
    s = jnp.sum(stats_ref[...], axis=0)
    sme = s[:, 0:_D]
    s1 = s[:, _D:2 * _D]
    sm = s[:, 256:257]
    s2 = s[:, 257:258]
    cnt = s[:, 258:259]
    c = sme / sm
    cs1 = jnp.sum(c * s1, axis=1, keepdims=True)
    cck = jnp.sum(c * c, axis=1, keepdims=True)
    intra = (s2 - 2.0 * cs1 + cnt * cck) / cnt
    loss_intra = jnp.sum(intra) / _K

    g = lax.dot_general(c, c, (((1,), (1,)), ((), ())),
                        preferred_element_type=jnp.float32)
    ccv = jnp.sum(c * c, axis=1)
    d2 = ccv[:, None] + ccv[None, :] - 2.0 * g
    pd = jnp.sqrt(jnp.maximum(d2, 0.0))
    q = jnp.sqrt(size_ref[0, :])
    qq = q[:, None] * q[None, :]
    ii = lax.broadcasted_iota(jnp.int32, (_K, _K), 0)
    jj = lax.broadcasted_iota(jnp.int32, (_K, _K), 1)
    off = ii != jj
    inter = jnp.sum(jnp.where(off, qq, 0.0) / jnp.where(off, pd, 1.0))
    loss_inter = _ALPHA * inter / (_K * (_K - 1))

    row = lax.broadcasted_iota(jnp.int32, (8, 128), 0)
    lane = lax.broadcasted_iota(jnp.int32, (8, 128), 1)
    vals = jnp.where(lane == 0, loss_intra + loss_inter,
                     jnp.where(lane == 1, loss_intra,
                               jnp.where(lane == 2, loss_inter, 0.0)))
    o_ref[...] = jnp.where(row == 0, vals, 0.0)


_tc_stage2 = pl.pallas_call(
    _tc_body,
    out_shape=jax.ShapeDtypeStruct((8, 128), jnp.float32),
)


def kernel(embeddings, labels, mass, size_map):
    zinit = jnp.zeros((_K, _COLS), jnp.float32)
    stats = _sc_stage1(embeddings.reshape(_N * _D), labels, mass, zinit)
    o = _tc_stage2(stats, size_map.reshape(1, _K))
    return (o[0, 0], o[0, 1], o[0, 2])


# speedup vs baseline: 7.5689x; 3.7303x over previous
import jax
import jax.numpy as jnp
from jax import lax
from jax.experimental import pallas as pl
from jax.experimental.pallas import tpu as pltpu
from jax.experimental.pallas import tpu_sc as plsc

_ALPHA = 0.1
_N = 32768
_D = 128
_K = 256
_NC = 2
_NS = 16
_NW = _NC * _NS
_TPW = _N // _NW
_CHUNK = 128
_NCHUNK = _TPW // _CHUNK
_COLS = 384
_ACC = _K * _COLS


def _vec_sqrt(x):
    i = lax.bitcast_convert_type(x, jnp.int32)
    i = jnp.int32(0x5F3759DF) - lax.shift_right_logical(i, 1)
    y = lax.bitcast_convert_type(i, jnp.float32)
    half = x * 0.5
    for _ in range(3):
        y = y * (1.5 - half * y * y)
    return jnp.where(x == 0.0, 0.0, x * y)


def _sc_body(emb, labels, mass, zinit, out, lab_v, mass_v, ebuf, acc):
    cid = lax.axis_index("c")
    sid = lax.axis_index("s")
    wid = sid * _NC + cid
    tok0 = wid * _TPW

    pltpu.sync_copy(zinit, acc)
    pltpu.sync_copy(labels.at[pl.ds(tok0, _TPW)], lab_v)
    pltpu.sync_copy(mass.at[pl.ds(tok0, _TPW)], mass_v)

    iota16 = lax.iota(jnp.int32, 16)

    def _prep(i, carry):
        sl = pl.ds(i * 16, 16)
        lab_v[sl] = lab_v[sl] * _COLS
        mass_v[sl] = _vec_sqrt(mass_v[sl])
        return carry
    lax.fori_loop(0, _TPW // 16, _prep, 0)

    cme = [iota16 + (j * 16) for j in range(_D // 16)]
    csq = iota16 + 272
    cmisc = iota16 + 288
    lane_lt8 = iota16 < 8

    def _chunk(ci, carry):
        pltpu.sync_copy(
            emb.at[pl.ds((tok0 + ci * _CHUNK) * _D, _CHUNK * _D)], ebuf)

        def _tok(t, c2):
            tg = jnp.full((16,), ci * _CHUNK + t, jnp.int32)
            labsplat = plsc.load_gather(lab_v, [tg])
            msplat = plsc.load_gather(mass_v, [tg])
            toff = t * _D
            sq = jnp.zeros((16,), jnp.float32)
            for j in range(_D // 16):
                ej = ebuf[pl.ds(toff + j * 16, 16)]
                plsc.addupdate_scatter(acc, [labsplat + cme[j]], ej * msplat)
                plsc.addupdate_scatter(acc, [labsplat + (cme[j] + _D)], ej)
                sq = sq + ej * ej
            plsc.addupdate_scatter(acc, [labsplat + csq], sq)
            misc = jnp.where(lane_lt8, msplat, 1.0)
            plsc.addupdate_scatter(acc, [labsplat + cmisc], misc)
            return c2

        lax.fori_loop(0, _CHUNK, _tok, 0)
        return carry

    lax.fori_loop(0, _NCHUNK, _chunk, 0)

    pltpu.sync_copy(acc, out.at[wid])


_sc_stage1 = pl.kernel(
    _sc_body,
    out_type=jax.ShapeDtypeStruct((_NW, _ACC), jnp.float32),
    mesh=plsc.VectorSubcoreMesh(
        core_axis_name="c", subcore_axis_name="s",
        num_cores=_NC, num_subcores=_NS),
    compiler_params=pltpu.CompilerParams(needs_layout_passes=False),
    scratch_types=[
        pltpu.VMEM((_TPW,), jnp.int32),
        pltpu.VMEM((_TPW,), jnp.float32),
        pltpu.VMEM((_CHUNK * _D,), jnp.float32),
        pltpu.VMEM((_ACC,), jnp.float32),
    ],
)


def _tc_body(stats_ref, size_ref, o_ref):
    s = jnp.sum(stats_ref[...], axis=0)
    sme = s[:, 0:_D]
    s1 = s[:, _D:2 * _D]
    s2 = jnp.sum(s[:, 272:288], axis=1, keepdims=True)
    sm = s[:, 288:289]
    cnt = s[:, 296:297]
    c = sme / sm
    cs1 = jnp.sum(c * s1, axis=1, keepdims=True)
    cck = jnp.sum(c * c, axis=1, keepdims=True)
    intra = (s2 - 2.0 * cs1 + cnt * cck) / cnt
    loss_intra = jnp.sum(intra) / _K

    g = lax.dot_general(c, c, (((1,), (1,)), ((), ())),
                        preferred_element_type=jnp.float32)
    ccv = jnp.sum(c * c, axis=1)
    d2 = ccv[:, None] + ccv[None, :] - 2.0 * g
    pd = jnp.sqrt(jnp.maximum(d2, 0.0))
    q = jnp.sqrt(size_ref[0, :])
    qq = q[:, None] * q[None, :]
    ii = lax.broadcasted_iota(jnp.int32, (_K, _K), 0)
    jj = lax.broadcasted_iota(jnp.int32, (_K, _K), 1)
    off = ii != jj
    inter = jnp.sum(jnp.where(off, qq, 0.0) / jnp.where(off, pd, 1.0))
    loss_inter = _ALPHA * inter / (_K * (_K - 1))

    row = lax.broadcasted_iota(jnp.int32, (8, 128), 0)
    lane = lax.broadcasted_iota(jnp.int32, (8, 128), 1)
    vals = jnp.where(lane == 0, loss_intra + loss_inter,
                     jnp.where(lane == 1, loss_intra,
                               jnp.where(lane == 2, loss_inter, 0.0)))
    o_ref[...] = jnp.where(row == 0, vals, 0.0)


_tc_stage2 = pl.pallas_call(
    _tc_body,
    out_shape=jax.ShapeDtypeStruct((8, 128), jnp.float32),
)


def kernel(embeddings, labels, mass, size_map):
    zinit = jnp.zeros((_ACC,), jnp.float32)
    stats = _sc_stage1(embeddings.reshape(_N * _D), labels, mass, zinit)
    o = _tc_stage2(stats.reshape(_NW, _K, _COLS), size_map.reshape(1, _K))
    return (o[0, 0], o[0, 1], o[0, 2])


# speedup vs baseline: 8.4104x; 1.1112x over previous
import jax
import jax.numpy as jnp
from jax import lax
from jax.experimental import pallas as pl
from jax.experimental.pallas import tpu as pltpu
from jax.experimental.pallas import tpu_sc as plsc

_ALPHA = 0.1
_N = 32768
_D = 128
_K = 256
_NC = 2
_NS = 16
_NW = _NC * _NS
_TPW = _N // _NW
_CHUNK = 128
_NCHUNK = _TPW // _CHUNK
_COLS = 384
_ACC = _K * _COLS


def _vec_sqrt(x):
    i = lax.bitcast_convert_type(x, jnp.int32)
    i = jnp.int32(0x5F3759DF) - lax.shift_right_logical(i, 1)
    y = lax.bitcast_convert_type(i, jnp.float32)
    half = x * 0.5
    for _ in range(3):
        y = y * (1.5 - half * y * y)
    return jnp.where(x == 0.0, 0.0, x * y)


def _sc_body(emb, labels, mass, zinit, out, lab_v, mass_v, ebuf, acc):
    cid = lax.axis_index("c")
    sid = lax.axis_index("s")
    wid = sid * _NC + cid
    tok0 = wid * _TPW

    pltpu.sync_copy(zinit, acc)
    pltpu.sync_copy(labels.at[pl.ds(tok0, _TPW)], lab_v)
    pltpu.sync_copy(mass.at[pl.ds(tok0, _TPW)], mass_v)

    iota16 = lax.iota(jnp.int32, 16)

    def _prep(i, carry):
        sl = pl.ds(i * 16, 16)
        mass_v[sl] = _vec_sqrt(mass_v[sl])
        return carry
    lax.fori_loop(0, _TPW // 16, _prep, 0)

    cme = [iota16 + (j * 16) for j in range(_D // 16)]
    csq = iota16 + 272
    cmisc = iota16 + 288
    lane_lt8 = iota16 < 8

    def _one_tok(ci, t):
        tg = jnp.full((16,), ci * _CHUNK + t, jnp.int32)
        labsplat = plsc.load_gather(lab_v, [tg])
        msplat = plsc.load_gather(mass_v, [tg])
        sq = jnp.zeros((16,), jnp.float32)
        for j in range(_D // 16):
            ej = ebuf[t, pl.ds(j * 16, 16)]
            plsc.addupdate_scatter(acc, [labsplat, cme[j]], ej * msplat)
            plsc.addupdate_scatter(acc, [labsplat, cme[j] + _D], ej)
            sq = sq + ej * ej
        plsc.addupdate_scatter(acc, [labsplat, csq], sq)
        misc = jnp.where(lane_lt8, msplat, 1.0)
        plsc.addupdate_scatter(acc, [labsplat, cmisc], misc)

    def _chunk(ci, carry):
        pltpu.sync_copy(emb.at[pl.ds(tok0 + ci * _CHUNK, _CHUNK), :], ebuf)

        def _tok4(t4, c2):
            for u in range(4):
                _one_tok(ci, t4 * 4 + u)
            return c2

        lax.fori_loop(0, _CHUNK // 4, _tok4, 0)
        return carry

    lax.fori_loop(0, _NCHUNK, _chunk, 0)

    pltpu.sync_copy(acc, out.at[wid])


_sc_stage1 = pl.kernel(
    _sc_body,
    out_type=jax.ShapeDtypeStruct((_NW, _K, _COLS), jnp.float32),
    mesh=plsc.VectorSubcoreMesh(
        core_axis_name="c", subcore_axis_name="s",
        num_cores=_NC, num_subcores=_NS),
    compiler_params=pltpu.CompilerParams(needs_layout_passes=False),
    scratch_types=[
        pltpu.VMEM((_TPW,), jnp.int32),
        pltpu.VMEM((_TPW,), jnp.float32),
        pltpu.VMEM((_CHUNK, _D), jnp.float32),
        pltpu.VMEM((_K, _COLS), jnp.float32),
    ],
)


def _tc_body(stats_ref, size_ref, o_ref):
    s = jnp.sum(stats_ref[...], axis=0)
    sme = s[:, 0:_D]
    s1 = s[:, _D:2 * _D]
    s2 = jnp.sum(s[:, 272:288], axis=1, keepdims=True)
    sm = s[:, 288:289]
    cnt = s[:, 296:297]
    c = sme / sm
    cs1 = jnp.sum(c * s1, axis=1, keepdims=True)
    cck = jnp.sum(c * c, axis=1, keepdims=True)
    intra = (s2 - 2.0 * cs1 + cnt * cck) / cnt
    loss_intra = jnp.sum(intra) / _K

    g = lax.dot_general(c, c, (((1,), (1,)), ((), ())),
                        preferred_element_type=jnp.float32)
    ccv = jnp.sum(c * c, axis=1)
    d2 = ccv[:, None] + ccv[None, :] - 2.0 * g
    pd = jnp.sqrt(jnp.maximum(d2, 0.0))
    q = jnp.sqrt(size_ref[0, :])
    qq = q[:, None] * q[None, :]
    ii = lax.broadcasted_iota(jnp.int32, (_K, _K), 0)
    jj = lax.broadcasted_iota(jnp.int32, (_K, _K), 1)
    off = ii != jj
    inter = jnp.sum(jnp.where(off, qq, 0.0) / jnp.where(off, pd, 1.0))
    loss_inter = _ALPHA * inter / (_K * (_K - 1))

    row = lax.broadcasted_iota(jnp.int32, (8, 128), 0)
    lane = lax.broadcasted_iota(jnp.int32, (8, 128), 1)
    vals = jnp.where(lane == 0, loss_intra + loss_inter,
                     jnp.where(lane == 1, loss_intra,
                               jnp.where(lane == 2, loss_inter, 0.0)))
    o_ref[...] = jnp.where(row == 0, vals, 0.0)


_tc_stage2 = pl.pallas_call(
    _tc_body,
    out_shape=jax.ShapeDtypeStruct((8, 128), jnp.float32),
)


def kernel(embeddings, labels, mass, size_map):
    zinit = jnp.zeros((_K, _COLS), jnp.float32)
    stats = _sc_stage1(embeddings, labels, mass, zinit)
    o = _tc_stage2(stats, size_map.reshape(1, _K))
    return (o[0, 0], o[0, 1], o[0, 2])
